# baseline (speedup 1.0000x reference)
import functools

import jax
import jax.numpy as jnp
from jax import lax
from jax.experimental import pallas as pl
from jax.experimental.pallas import tpu as pltpu
from jax.experimental.pallas import tpu_sc as plsc

_INFO = plsc.get_sparse_core_info()
_NC, _NS = _INFO.num_cores, _INFO.num_subcores
_NW = _NC * _NS

_TOK = 4


@functools.cache
def _build(T, batch, v, d):
    rows = _TOK * batch
    n_active = T // _TOK
    mesh = plsc.VectorSubcoreMesh(core_axis_name="c", subcore_axis_name="s")

    @functools.partial(
        pl.kernel,
        out_type=jax.ShapeDtypeStruct((T, batch, d), jnp.float32),
        mesh=mesh,
        scratch_types=[
            pltpu.VMEM((rows,), jnp.int32),
            pltpu.VMEM((rows, d), jnp.float32),
            pltpu.SemaphoreType.DMA,
            pltpu.SemaphoreType.DMA,
        ],
    )
    def gather_kernel(iv_hbm, table_hbm, out_hbm, idx_v, rows_v, sem, sem_w):
        wid = lax.axis_index("s") * _NC + lax.axis_index("c")

        @pl.when(wid < n_active)
        def _():
            t0 = wid * _TOK
            pltpu.sync_copy(iv_hbm.at[pl.ds(t0 * batch, rows)], idx_v)
            half = rows // 2
            gathers = [
                pltpu.async_copy(
                    table_hbm.at[idx_v.at[pl.ds(g * half, half)]],
                    rows_v.at[pl.ds(g * half, half)],
                    sem,
                )
                for g in range(2)
            ]
            writes = []
            for g in range(2):
                gathers[g].wait()
                for k in range(2 * g, 2 * g + 2):
                    writes.append(
                        pltpu.async_copy(
                            rows_v.at[pl.ds(k * batch, batch)],
                            out_hbm.at[t0 + k],
                            sem_w,
                        )
                    )
            for w in writes:
                w.wait()

    return gather_kernel


def kernel(indices, embedding):
    batch, t = indices.shape
    v, d = embedding.shape
    iv = indices.astype(jnp.int32).T.reshape(t * batch)
    out = _build(t, batch, v, d)(iv, embedding)
    return out.transpose(1, 0, 2)


# speedup vs baseline: 1.0333x; 1.0333x over previous
import functools

import jax
import jax.numpy as jnp
from jax import lax
from jax.experimental import pallas as pl
from jax.experimental.pallas import tpu as pltpu
from jax.experimental.pallas import tpu_sc as plsc

_INFO = plsc.get_sparse_core_info()
_NC, _NS = _INFO.num_cores, _INFO.num_subcores
_NW = _NC * _NS

_TOK = 4


@functools.cache
def _build(T, batch, v, d):
    rows = _TOK * batch
    n_active = T // _TOK
    mesh = plsc.VectorSubcoreMesh(core_axis_name="c", subcore_axis_name="s")

    @functools.partial(
        pl.kernel,
        out_type=jax.ShapeDtypeStruct((T, batch, d), jnp.float32),
        mesh=mesh,
        scratch_types=[
            pltpu.VMEM((rows,), jnp.int32),
            pltpu.VMEM((rows, d), jnp.float32),
            pltpu.SemaphoreType.DMA,
            pltpu.SemaphoreType.DMA,
        ],
    )
    def gather_kernel(iv_hbm, table_hbm, out_hbm, idx_v, rows_v, sem, sem_w):
        wid = lax.axis_index("s") * _NC + lax.axis_index("c")

        @pl.when(wid < n_active)
        def _():
            t0 = wid * _TOK
            pltpu.sync_copy(iv_hbm.at[pl.ds(t0 * batch, rows)], idx_v)
            pltpu.async_copy(table_hbm.at[idx_v], rows_v, sem).wait()
            writes = [
                pltpu.async_copy(
                    rows_v.at[pl.ds(k * batch, batch)],
                    out_hbm.at[t0 + k],
                    sem_w,
                )
                for k in range(_TOK)
            ]
            for w in writes:
                w.wait()

    return gather_kernel


def kernel(indices, embedding):
    batch, t = indices.shape
    v, d = embedding.shape
    iv = indices.astype(jnp.int32).T.reshape(t * batch)
    out = _build(t, batch, v, d)(iv, embedding)
    return out.transpose(1, 0, 2)
